# Initial kernel scaffold; baseline (speedup 1.0000x reference)
import jax
import jax.numpy as jnp
from jax.experimental import pallas as pl


def kernel(input_ids, attention_mask, emb, W, b):
    raise NotImplementedError("write your pallas kernel here")


# speedup vs baseline: 16.5742x; 16.5742x over previous
import functools

import jax
import jax.numpy as jnp
from jax import lax
from jax.experimental import pallas as pl
from jax.experimental.pallas import tpu as pltpu
from jax.experimental.pallas import tpu_sc as plsc

_B = 4096
_L = 200
_V = 100000
_H = 32
_NL = 6
_D = 16
_NW = 32
_SPW = _B // _NW
_CSEQ = 16
_NCH = _SPW // _CSEQ
_CIDX = _CSEQ * _L
_IW = 128
_NSTR = _CIDX // _IW

_TC_ROWS = 4000


def _tc_table_body(emb_ref, w_ref, out_ref):
    out_ref[...] = jnp.dot(
        emb_ref[...], w_ref[...] * (1.0 / _L), preferred_element_type=jnp.float32
    )


def _make_table(emb, w_pad):
    return pl.pallas_call(
        _tc_table_body,
        grid=(_V // _TC_ROWS,),
        in_specs=[
            pl.BlockSpec((_TC_ROWS, _H), lambda i: (i, 0)),
            pl.BlockSpec((_H, _D), lambda i: (0, 0)),
        ],
        out_specs=pl.BlockSpec((_TC_ROWS, _D), lambda i: (i, 0)),
        out_shape=jax.ShapeDtypeStruct((_V, _D), jnp.float32),
    )(emb, w_pad)


def _sc_pool(ids_flat, embw, b16):
    mesh = plsc.VectorSubcoreMesh(core_axis_name="c", subcore_axis_name="s")

    @functools.partial(
        pl.kernel,
        out_type=jax.ShapeDtypeStruct((_B, _D), jnp.float32),
        mesh=mesh,
        scratch_types=[
            pltpu.VMEM((_SPW * _L // _IW, _IW), jnp.int32),
            pltpu.VMEM((2, _CIDX, _D), jnp.float32),
            pltpu.VMEM((_SPW, _D), jnp.float32),
            pltpu.VMEM((_D,), jnp.float32),
            pltpu.SemaphoreType.DMA,
            pltpu.SemaphoreType.DMA,
        ],
        compiler_params=pltpu.CompilerParams(use_tc_tiling_on_sc=False),
    )
    def k(ids_hbm, embw_hbm, b16_hbm, out_hbm, idx_v, rows_v, pool_v, b_v,
          sem0, sem1):
        cid = lax.axis_index("c")
        sid = lax.axis_index("s")
        wid = cid * 16 + sid
        seq0 = pl.multiple_of(wid * _SPW, 8)
        row0 = pl.multiple_of(wid * (_SPW * _L // _IW), 8)

        pltpu.sync_copy(ids_hbm.at[pl.ds(row0, _SPW * _L // _IW), :], idx_v)
        pltpu.sync_copy(b16_hbm, b_v)

        def gather(c, j, buf, sem):
            return pltpu.make_async_copy(
                embw_hbm.at[idx_v.at[c * _NSTR + j]],
                rows_v.at[buf, pl.ds(j * _IW, _IW), :],
                sem,
            )

        def start(c, buf, sem):
            def fire(j, carry):
                gather(c, j, buf, sem).start()
                return carry

            lax.fori_loop(0, _NSTR, fire, 0)

        def finish_and_reduce(c, buf, sem):
            def drain(j, carry):
                gather(c, j, buf, sem).wait()
                return carry

            lax.fori_loop(0, _NSTR, drain, 0)
            bias = b_v[...]

            def seq_body(s, carry):
                rbase = s * _L
                zero = jnp.zeros((_D,), jnp.float32)

                def red(kk, accs):
                    a0, a1, a2, a3 = accs
                    r = rbase + kk * 8
                    a0 = a0 + rows_v[buf, r, :]
                    a1 = a1 + rows_v[buf, r + 1, :]
                    a2 = a2 + rows_v[buf, r + 2, :]
                    a3 = a3 + rows_v[buf, r + 3, :]
                    a0 = a0 + rows_v[buf, r + 4, :]
                    a1 = a1 + rows_v[buf, r + 5, :]
                    a2 = a2 + rows_v[buf, r + 6, :]
                    a3 = a3 + rows_v[buf, r + 7, :]
                    return (a0, a1, a2, a3)

                a0, a1, a2, a3 = lax.fori_loop(
                    0, _L // 8, red, (zero, zero, zero, zero)
                )
                pool_v[c * _CSEQ + s, :] = ((a0 + a1) + (a2 + a3)) + bias
                return carry

            lax.fori_loop(0, _CSEQ, seq_body, 0)

        sems = (sem0, sem1)
        start(0, 0, sems[0])
        for c in range(_NCH):
            buf = c & 1
            if c + 1 < _NCH:
                start(c + 1, 1 - buf, sems[1 - buf])
            finish_and_reduce(c, buf, sems[buf])

        pltpu.sync_copy(pool_v, out_hbm.at[pl.ds(seq0, _SPW), :])

    return k(ids_flat, embw, b16)


def kernel(input_ids, attention_mask, emb, W, b):
    del attention_mask
    ids_flat = input_ids.reshape(_B * _L // _IW, _IW).astype(jnp.int32)
    w_pad = jnp.zeros((_H, _D), jnp.float32).at[:, :_NL].set(W)
    b16 = jnp.zeros((_D,), jnp.float32).at[:_NL].set(b)
    embw = _make_table(emb, w_pad)
    out = _sc_pool(ids_flat, embw, b16)
    return out[:, :_NL]


# speedup vs baseline: 19.3185x; 1.1656x over previous
import functools

import jax
import jax.numpy as jnp
from jax import lax
from jax.experimental import pallas as pl
from jax.experimental.pallas import tpu as pltpu
from jax.experimental.pallas import tpu_sc as plsc

_B = 4096
_L = 200
_V = 100000
_H = 32
_NL = 6
_D = 16
_NW = 32
_SPW = _B // _NW
_CSEQ = 16
_NCH = _SPW // _CSEQ
_CIDX = _CSEQ * _L
_IW = 128
_NSTR = _CIDX // _IW

_PK = 128 // _D
_SL = 16384
_VP = _SL * _PK
_TC_OUT = 512
_TC_GRID = _SL // _TC_OUT
_LAST_BLK = (_V - 1) // _TC_OUT


def _tc_table_body(*refs):
    w = refs[_PK][...] * (1.0 / _L)
    out_ref = refs[_PK + 1]
    out_ref[...] = jnp.concatenate(
        [
            jnp.dot(refs[k][...], w, preferred_element_type=jnp.float32)
            for k in range(_PK)
        ],
        axis=1,
    )


def _make_table(emb, w_pad):
    in_specs = [
        pl.BlockSpec((_TC_OUT, _H), lambda i, k=k: (jnp.minimum(k * _TC_GRID + i, _LAST_BLK), 0))
        for k in range(_PK)
    ]
    in_specs.append(pl.BlockSpec((_H, _D), lambda i: (0, 0)))
    return pl.pallas_call(
        _tc_table_body,
        grid=(_TC_GRID,),
        in_specs=in_specs,
        out_specs=pl.BlockSpec((_TC_OUT, 128), lambda i: (i, 0)),
        out_shape=jax.ShapeDtypeStruct((_SL, 128), jnp.float32),
    )(*([emb] * _PK), w_pad)


def _tc_ids_body(ids_ref, out_ref):
    v = ids_ref[...]
    out_ref[...] = ((v & (_SL - 1)) << 3) | (v >> 14)


def _remap_ids(ids_r):
    n = ids_r.shape[0]
    blk = n // 10
    return pl.pallas_call(
        _tc_ids_body,
        grid=(10,),
        in_specs=[pl.BlockSpec((blk, _IW), lambda i: (i, 0))],
        out_specs=pl.BlockSpec((blk, _IW), lambda i: (i, 0)),
        out_shape=jax.ShapeDtypeStruct((n, _IW), jnp.int32),
    )(ids_r)


def _sc_pool(ids_flat, embw, b16):
    mesh = plsc.VectorSubcoreMesh(core_axis_name="c", subcore_axis_name="s")

    @functools.partial(
        pl.kernel,
        out_type=jax.ShapeDtypeStruct((_B, _D), jnp.float32),
        mesh=mesh,
        scratch_types=[
            pltpu.VMEM((_SPW * _L // _IW, _IW), jnp.int32),
            pltpu.VMEM((2, _CIDX, _D), jnp.float32),
            pltpu.VMEM((_SPW, _D), jnp.float32),
            pltpu.VMEM((_D,), jnp.float32),
            pltpu.SemaphoreType.DMA,
            pltpu.SemaphoreType.DMA,
        ],
        compiler_params=pltpu.CompilerParams(use_tc_tiling_on_sc=False),
    )
    def k(ids_hbm, embw_hbm, b16_hbm, out_hbm, idx_v, rows_v, pool_v,
          b_v, sem0, sem1):
        cid = lax.axis_index("c")
        sid = lax.axis_index("s")
        wid = cid * 16 + sid
        seq0 = pl.multiple_of(wid * _SPW, 8)
        row0 = pl.multiple_of(wid * (_SPW * _L // _IW), 8)

        pltpu.sync_copy(ids_hbm.at[pl.ds(row0, _SPW * _L // _IW), :], idx_v)
        pltpu.sync_copy(b16_hbm, b_v)

        def gather(c, j, buf, sem):
            return pltpu.make_async_copy(
                embw_hbm.at[idx_v.at[c * _NSTR + j]],
                rows_v.at[buf, pl.ds(j * _IW, _IW), :],
                sem,
            )

        def start(c, buf, sem):
            def fire(j, carry):
                gather(c, j, buf, sem).start()
                return carry

            lax.fori_loop(0, _NSTR, fire, 0)

        def finish_and_reduce(c, buf, sem):
            def drain(j, carry):
                gather(c, j, buf, sem).wait()
                return carry

            lax.fori_loop(0, _NSTR, drain, 0)
            bias = b_v[...]

            def seq_body(s, carry):
                rbase = s * _L
                zero = jnp.zeros((_D,), jnp.float32)

                def red(kk, accs):
                    r = rbase + kk * 40
                    accs = list(accs)
                    for u in range(40):
                        accs[u % 8] = accs[u % 8] + rows_v[buf, r + u, :]
                    return tuple(accs)

                accs = lax.fori_loop(0, _L // 40, red, (zero,) * 8)
                t0 = (accs[0] + accs[1]) + (accs[2] + accs[3])
                t1 = (accs[4] + accs[5]) + (accs[6] + accs[7])
                pool_v[c * _CSEQ + s, :] = (t0 + t1) + bias
                return carry

            lax.fori_loop(0, _CSEQ, seq_body, 0)

        sems = (sem0, sem1)
        start(0, 0, sems[0])
        for c in range(_NCH):
            buf = c & 1
            if c + 1 < _NCH:
                start(c + 1, 1 - buf, sems[1 - buf])
            finish_and_reduce(c, buf, sems[buf])

        pltpu.sync_copy(pool_v, out_hbm.at[pl.ds(seq0, _SPW), :])

    return k(ids_flat, embw, b16)


def kernel(input_ids, attention_mask, emb, W, b):
    del attention_mask
    ids_r = input_ids.reshape(_B * _L // _IW, _IW).astype(jnp.int32)
    ids_m = _remap_ids(ids_r)
    w_pad = jnp.zeros((_H, _D), jnp.float32).at[:, :_NL].set(W)
    b16 = jnp.zeros((_D,), jnp.float32).at[:_NL].set(b)
    embw = _make_table(emb, w_pad).reshape(_VP, _D)
    out = _sc_pool(ids_m, embw, b16)
    return out[:, :_NL]


# speedup vs baseline: 24.7215x; 1.2797x over previous
import functools

import jax
import jax.numpy as jnp
from jax import lax
from jax.experimental import pallas as pl
from jax.experimental.pallas import tpu as pltpu
from jax.experimental.pallas import tpu_sc as plsc

_B = 4096
_L = 200
_V = 100000
_H = 32
_NL = 6
_D = 16
_NW = 32
_SPW = _B // _NW
_CPOS = 25
_NCH = _L // _CPOS
_IW = 128

_PK = 128 // _D
_SL = 16384
_VP = _SL * _PK
_TC_OUT = 512
_TC_GRID = _SL // _TC_OUT
_LAST_BLK = (_V - 1) // _TC_OUT


def _tc_table_body(*refs):
    w = refs[_PK][...] * (1.0 / _L)
    out_ref = refs[_PK + 1]
    dn = (((0,), (0,)), ((), ()))
    out_ref[...] = jnp.concatenate(
        [
            lax.dot_general(refs[k][...], w, dn, preferred_element_type=jnp.float32)
            for k in range(_PK)
        ],
        axis=1,
    )


def _make_table(emb_t, w_pad):
    in_specs = [
        pl.BlockSpec(
            (_H, _TC_OUT),
            lambda i, k=k: (0, jnp.minimum(k * _TC_GRID + i, _LAST_BLK)),
        )
        for k in range(_PK)
    ]
    in_specs.append(pl.BlockSpec((_H, _D), lambda i: (0, 0)))
    return pl.pallas_call(
        _tc_table_body,
        grid=(_TC_GRID,),
        in_specs=in_specs,
        out_specs=pl.BlockSpec((_TC_OUT, 128), lambda i: (i, 0)),
        out_shape=jax.ShapeDtypeStruct((_SL, 128), jnp.float32),
    )(*([emb_t] * _PK), w_pad)


def _tc_ids_body(ids_ref, out_ref):
    v = ids_ref[...]
    out_ref[...] = ((v & (_SL - 1)) << 3) | (v >> 14)


def _remap_ids(ids_t):
    return pl.pallas_call(
        _tc_ids_body,
        grid=(5,),
        in_specs=[pl.BlockSpec((_L // 5, _B), lambda i: (i, 0))],
        out_specs=pl.BlockSpec((_L // 5, _B), lambda i: (i, 0)),
        out_shape=jax.ShapeDtypeStruct((_L, _B), jnp.int32),
    )(ids_t)


def _sc_pool(ids_t, embw, b16):
    mesh = plsc.VectorSubcoreMesh(core_axis_name="c", subcore_axis_name="s")

    @functools.partial(
        pl.kernel,
        out_type=jax.ShapeDtypeStruct((_B, _D), jnp.float32),
        mesh=mesh,
        scratch_types=[
            pltpu.VMEM((_L, _SPW), jnp.int32),
            pltpu.VMEM((2, _CPOS * _IW, _D), jnp.float32),
            pltpu.VMEM((_SPW, _D), jnp.float32),
            pltpu.VMEM((_D,), jnp.float32),
            pltpu.SemaphoreType.DMA,
            pltpu.SemaphoreType.DMA,
        ],
        compiler_params=pltpu.CompilerParams(use_tc_tiling_on_sc=False),
    )
    def k(ids_hbm, embw_hbm, b16_hbm, out_hbm, idx_v, rows_v, pool_v,
          b_v, sem0, sem1):
        cid = lax.axis_index("c")
        sid = lax.axis_index("s")
        wid = cid * 16 + sid
        seq0 = pl.multiple_of(wid * _SPW, _SPW)

        pltpu.sync_copy(ids_hbm.at[:, pl.ds(seq0, _SPW)], idx_v)
        pltpu.sync_copy(b16_hbm, b_v)

        def gather(c, j, buf, sem):
            return pltpu.make_async_copy(
                embw_hbm.at[idx_v.at[c * _CPOS + j]],
                rows_v.at[buf, pl.ds(j * _IW, _IW), :],
                sem,
            )

        def start(c, buf, sem):
            def fire(j, carry):
                gather(c, j, buf, sem).start()
                return carry

            lax.fori_loop(0, _CPOS, fire, 0)

        def finish_and_reduce(c, buf, sem):
            def drain(j, carry):
                gather(c, j, buf, sem).wait()
                return carry

            lax.fori_loop(0, _CPOS, drain, 0)
            bias = b_v[...]

            def seq_body(s, carry):
                accs = [jnp.zeros((_D,), jnp.float32)] * 5
                for j in range(_CPOS):
                    accs[j % 5] = accs[j % 5] + rows_v[buf, j * _IW + s, :]
                acc = ((accs[0] + accs[1]) + (accs[2] + accs[3])) + accs[4]
                if c == 0:
                    pool_v[s, :] = acc + bias
                else:
                    pool_v[s, :] = pool_v[s, :] + acc
                return carry

            lax.fori_loop(0, _SPW, seq_body, 0)

        sems = (sem0, sem1)
        start(0, 0, sems[0])
        for c in range(_NCH):
            buf = c & 1
            if c + 1 < _NCH:
                start(c + 1, 1 - buf, sems[1 - buf])
            finish_and_reduce(c, buf, sems[buf])

        pltpu.sync_copy(pool_v, out_hbm.at[pl.ds(seq0, _SPW), :])

    return k(ids_t, embw, b16)


def kernel(input_ids, attention_mask, emb, W, b):
    del attention_mask
    ids_t = input_ids.T.astype(jnp.int32)
    ids_m = _remap_ids(ids_t)
    w_pad = jnp.zeros((_H, _D), jnp.float32).at[:, :_NL].set(W)
    b16 = jnp.zeros((_D,), jnp.float32).at[:_NL].set(b)
    embw = _make_table(emb.T, w_pad).reshape(_VP, _D)
    out = _sc_pool(ids_m, embw, b16)
    return out[:, :_NL]


# speedup vs baseline: 43.0483x; 1.7413x over previous
import functools

import jax
import jax.numpy as jnp
from jax import lax
from jax.experimental import pallas as pl
from jax.experimental.pallas import tpu as pltpu
from jax.experimental.pallas import tpu_sc as plsc

_B = 4096
_L = 200
_V = 100000
_H = 32
_NL = 6
_D = 16
_NW = 32
_SPW = _B // _NW
_CPOS = 25
_NCH = _L // _CPOS
_IW = 128

_PK = 128 // _D
_SL = 12800
_VP = _SL * _PK
_TC_OUT = 3200
_TC_GRID = _SL // _TC_OUT


def _tc_table_body(*refs):
    wp = jnp.pad(refs[_PK][...], ((0, 0), (0, _D - _NL))) * (1.0 / _L)
    wbig = jnp.concatenate(
        [
            jnp.pad(wp, ((0, 0), (k * _D, 128 - _D - k * _D)))
            for k in range(_PK)
        ],
        axis=0,
    )
    out_ref = refs[_PK + 1]
    e_cat = jnp.concatenate([refs[k][...] for k in range(_PK)], axis=0)
    dn = (((0,), (0,)), ((), ()))
    out_ref[...] = lax.dot_general(
        e_cat, wbig, dn, preferred_element_type=jnp.float32
    )


def _make_table(emb_t, w_pad):
    in_specs = [
        pl.BlockSpec((_H, _TC_OUT), lambda i, k=k: (0, k * _TC_GRID + i))
        for k in range(_PK)
    ]
    in_specs.append(pl.BlockSpec((_H, _NL), lambda i: (0, 0)))
    return pl.pallas_call(
        _tc_table_body,
        grid=(_TC_GRID,),
        in_specs=in_specs,
        out_specs=pl.BlockSpec((_TC_OUT, 128), lambda i: (i, 0)),
        out_shape=jax.ShapeDtypeStruct((_SL, 128), jnp.float32),
        compiler_params=pltpu.CompilerParams(fuse_transposed_lhs_in_matmul=True),
    )(*([emb_t] * _PK), w_pad)


def _sc_pool(ids4, embw, b_raw):
    mesh = plsc.VectorSubcoreMesh(core_axis_name="c", subcore_axis_name="s")

    @functools.partial(
        pl.kernel,
        out_type=jax.ShapeDtypeStruct((_B, _D), jnp.float32),
        mesh=mesh,
        scratch_types=[
            pltpu.VMEM((_L // 8, 8, _SPW), jnp.int32),
            pltpu.VMEM((2, _CPOS * _IW, _D), jnp.float32),
            pltpu.VMEM((_SPW, _D), jnp.float32),
            pltpu.VMEM((_D,), jnp.float32),
            pltpu.SemaphoreType.DMA,
            pltpu.SemaphoreType.DMA,
        ],
        compiler_params=pltpu.CompilerParams(use_tc_tiling_on_sc=False),
    )
    def k(ids_hbm, embw_hbm, b_hbm, out_hbm, idx_v, rows_v, pool_v,
          b_v, sem0, sem1):
        cid = lax.axis_index("c")
        sid = lax.axis_index("s")
        wid = cid * 16 + sid
        seq0 = pl.multiple_of(wid * _SPW, _SPW)

        b_v[...] = jnp.zeros((_D,), jnp.float32)
        pltpu.sync_copy(b_hbm, b_v.at[pl.ds(0, _NL)])

        def remap(p, carry):
            row = idx_v.at[p // 8, p % 8]
            for u in range(_SPW // 16):
                v = row[pl.ds(u * 16, 16)]
                q = ((v.astype(jnp.float32) + 0.5) * (1.0 / _SL)).astype(
                    jnp.int32
                )
                row[pl.ds(u * 16, 16)] = ((v - q * _SL) << 3) | q
            return carry

        def gather(c, j, buf, sem):
            p = c * _CPOS + j
            return pltpu.make_async_copy(
                embw_hbm.at[idx_v.at[p // 8, p % 8]],
                rows_v.at[buf, pl.ds(j * _IW, _IW), :],
                sem,
            )

        def start(c, buf, sem):
            lax.fori_loop(c * _CPOS, (c + 1) * _CPOS, remap, 0)

            def fire(j, carry):
                gather(c, j, buf, sem).start()
                return carry

            lax.fori_loop(0, _CPOS, fire, 0)

        def finish_and_reduce(c, buf, sem):
            def drain(j, carry):
                gather(c, j, buf, sem).wait()
                return carry

            lax.fori_loop(0, _CPOS, drain, 0)

            def seq_body(s, carry):
                accs = [jnp.zeros((_D,), jnp.float32)] * 5
                for j in range(_CPOS):
                    accs[j % 5] = accs[j % 5] + rows_v[buf, j * _IW + s, :]
                acc = ((accs[0] + accs[1]) + (accs[2] + accs[3])) + accs[4]
                pool_v[s, :] = pool_v[s, :] + acc
                return carry

            lax.fori_loop(0, _SPW, seq_body, 0)

        pltpu.sync_copy(ids_hbm.at[pl.ds(0, 4), wid, :, :],
                        idx_v.at[pl.ds(0, 4)])
        start(0, 0, sem0)
        pltpu.sync_copy(ids_hbm.at[pl.ds(4, _L // 8 - 4), wid, :, :],
                        idx_v.at[pl.ds(4, _L // 8 - 4)])
        bias = b_v[...]

        def pool_init(s, carry):
            pool_v[s, :] = bias
            return carry

        lax.fori_loop(0, _SPW, pool_init, 0)

        def pair(cc, carry):
            c = cc * 2
            start(c + 1, 1, sem1)
            finish_and_reduce(c, 0, sem0)

            @pl.when(cc < _NCH // 2 - 1)
            def _():
                start(c + 2, 0, sem0)

            finish_and_reduce(c + 1, 1, sem1)
            return carry

        lax.fori_loop(0, _NCH // 2, pair, 0)

        pltpu.sync_copy(pool_v, out_hbm.at[pl.ds(seq0, _SPW), :])

    return k(ids4, embw, b_raw)


def kernel(input_ids, attention_mask, emb, W, b):
    del attention_mask
    ids4 = (
        input_ids.T.astype(jnp.int32)
        .reshape(_L // 8, 8, _NW, _IW)
        .transpose(0, 2, 1, 3)
    )
    embw = _make_table(emb.T, W).reshape(_VP, _D)
    out = _sc_pool(ids4, embw, b)
    return out[:, :_NL]
